# baseline (speedup 1.0000x reference)
import functools

import jax
import jax.numpy as jnp
from jax import lax
from jax.experimental import pallas as pl
from jax.experimental.pallas import tpu as pltpu
from jax.experimental.pallas import tpu_sc as plsc

B = 16384
N_NUM = 13
N_CAT = 26
VOCAB = 100000
D = 32
N_TOK = 1 + N_NUM + N_CAT
D_OUT = N_TOK * D
CAT_COL0 = (1 + N_NUM) * D

NG = 7
NC, NS = 2, 16
NW = NC * NS
CHUNK = 1024
N_CHUNKS = B // CHUNK
N_ITEMS = N_CAT * N_CHUNKS
ITEMS_PER_W = N_ITEMS // NW
SUB = 128
NSUB = CHUNK // SUB

VWIN = 2048
VWINS = -(-VOCAB // VWIN)
VPAD = VWINS * VWIN
R_ROWS = N_CAT * VPAD // 4


def _repack_body(t_ref, eye_ref, out_ref):
    x = t_ref[...][0]
    xt = jax.lax.dot_general(
        x,
        eye_ref[...],
        dimension_numbers=(((0,), (0,)), ((), ())),
        preferred_element_type=jnp.float32,
    )
    y = xt.reshape(VWIN // 4, 4, D)
    for a in range(4):
        out_ref[:, pl.ds(a * D, D)] = y[:, a, :]


def _repack(tabT):
    grid = (N_CAT, VWINS)
    return pl.pallas_call(
        _repack_body,
        grid=grid,
        in_specs=[
            pl.BlockSpec((1, D, VWIN), lambda j, v: (j, 0, v)),
            pl.BlockSpec((D, D), lambda j, v: (0, 0)),
        ],
        out_specs=pl.BlockSpec(
            (VWIN // 4, 128), lambda j, v: (j * VWINS + v, 0)
        ),
        out_shape=jax.ShapeDtypeStruct((R_ROWS, 128), jnp.float32),
        compiler_params=pltpu.CompilerParams(
            dimension_semantics=("parallel", "parallel"),
        ),
    )(tabT, jnp.eye(D, dtype=jnp.float32))


def _sc_gather(table_flat, idx3):
    mesh = plsc.VectorSubcoreMesh(core_axis_name="c", subcore_axis_name="s")

    @functools.partial(
        pl.kernel,
        out_type=[jax.ShapeDtypeStruct((B, 128), jnp.float32) for _ in range(NG)],
        mesh=mesh,
        scratch_types=[
            pltpu.VMEM((NSUB, SUB), jnp.int32),
            pltpu.VMEM((CHUNK, D), jnp.float32),
            pltpu.SemaphoreType.DMA,
        ],
        compiler_params=pltpu.CompilerParams(use_tc_tiling_on_sc=False),
    )
    def k(table_hbm, idx_hbm, *refs):
        g_refs = refs[:NG]
        idx_v, tmp_v, gsem = refs[NG:]
        wid = lax.axis_index("s") * NC + lax.axis_index("c")

        @pl.loop(0, ITEMS_PER_W)
        def _(it):
            item = wid * ITEMS_PER_W + it
            j = item // N_CHUNKS
            c = item - j * N_CHUNKS
            pltpu.sync_copy(idx_hbm.at[item], idx_v)
            cps = [
                pltpu.async_copy(
                    table_hbm.at[idx_v.at[s]],
                    tmp_v.at[pl.ds(s * SUB, SUB)],
                    gsem,
                )
                for s in range(NSUB)
            ]
            for cp in cps:
                cp.wait()
            gi = j // 4
            f = j - gi * 4
            dst = [
                g.at[pl.ds(c * CHUNK, CHUNK), pl.ds(f * D, D)] for g in g_refs
            ]
            for gg in range(NG):
                @pl.when(gi == gg)
                def _():
                    pltpu.sync_copy(tmp_v, dst[gg])

    return k(table_flat, idx3)


def _assemble_body(xp_ref, s_ref, w2_ref, b2_ref, *refs):
    g_refs = refs[:NG]
    out_ref = refs[NG]
    num = (
        jnp.dot(xp_ref[...], s_ref[...], preferred_element_type=jnp.float32)
        * w2_ref[...]
        + b2_ref[...]
    )
    parts = [num] + [g_refs[gi][...] for gi in range(NG - 1)]
    parts.append(g_refs[NG - 1][...][:, : 2 * D])
    out_ref[...] = jnp.concatenate(parts, axis=1)


def _assemble(xp, sel, w2, b2, gs):
    bb = 512
    return pl.pallas_call(
        _assemble_body,
        grid=(B // bb,),
        in_specs=[
            pl.BlockSpec((bb, 16), lambda i: (i, 0)),
            pl.BlockSpec((16, CAT_COL0), lambda i: (0, 0)),
            pl.BlockSpec((1, CAT_COL0), lambda i: (0, 0)),
            pl.BlockSpec((1, CAT_COL0), lambda i: (0, 0)),
        ]
        + [pl.BlockSpec((bb, 128), lambda i: (i, 0)) for _ in range(NG)],
        out_specs=pl.BlockSpec((bb, D_OUT), lambda i: (i, 0)),
        out_shape=jax.ShapeDtypeStruct((B, D_OUT), jnp.float32),
        compiler_params=pltpu.CompilerParams(
            dimension_semantics=("arbitrary",),
        ),
    )(xp, sel, w2, b2, *gs)


def kernel(X_num, X_cat, num_weight, num_bias, cat_tables):
    offs = (jnp.arange(N_CAT, dtype=jnp.int32) * VPAD)[None, :]
    xo = X_cat.astype(jnp.int32) + offs
    idx3 = (
        xo.reshape(N_CHUNKS, NSUB, SUB, N_CAT)
        .transpose(3, 0, 1, 2)
        .reshape(N_ITEMS, NSUB, SUB)
    )
    tabT = cat_tables.transpose(0, 2, 1)
    table_flat = _repack(tabT).reshape(N_CAT * VPAD, D)
    gs = _sc_gather(table_flat, idx3)

    xp = jnp.concatenate(
        [jnp.zeros((B, 1), jnp.float32), X_num, jnp.zeros((B, 2), jnp.float32)],
        axis=1,
    )
    tok_of_col = jnp.arange(CAT_COL0, dtype=jnp.int32) // D
    sel = (tok_of_col[None, :] == jnp.arange(16, dtype=jnp.int32)[:, None]).astype(
        jnp.float32
    )
    w2 = jnp.concatenate([jnp.zeros((D,), jnp.float32), num_weight.reshape(-1)])[
        None, :
    ]
    b2 = jnp.concatenate([jnp.zeros((D,), jnp.float32), num_bias.reshape(-1)])[
        None, :
    ]
    return _assemble(xp, sel, w2, b2, gs)


# speedup vs baseline: 1.0324x; 1.0324x over previous
import functools

import jax
import jax.numpy as jnp
from jax import lax
from jax.experimental import pallas as pl
from jax.experimental.pallas import tpu as pltpu
from jax.experimental.pallas import tpu_sc as plsc

B = 16384
N_NUM = 13
N_CAT = 26
VOCAB = 100000
D = 32
N_TOK = 1 + N_NUM + N_CAT
D_OUT = N_TOK * D
CAT_COL0 = (1 + N_NUM) * D

NC, NS = 2, 16
NW = NC * NS
WROWS = B // NW
SUB = 128
NSUB = WROWS // SUB
HCH = 256
T_ROWS = N_CAT * VOCAB


def _sc_gather(table_pad, idx4):
    mesh = plsc.VectorSubcoreMesh(core_axis_name="c", subcore_axis_name="s")

    @functools.partial(
        pl.kernel,
        out_type=[
            jax.ShapeDtypeStruct((B, 128), jnp.float32) for _ in range(N_CAT)
        ],
        mesh=mesh,
        scratch_types=[
            pltpu.VMEM((N_CAT, NSUB, SUB), jnp.int32),
            pltpu.VMEM((2, HCH, 128), jnp.float32),
            pltpu.SemaphoreType.DMA,
            pltpu.SemaphoreType.DMA,
        ],
    )
    def k(table_hbm, idx_hbm, *refs):
        slabs = refs[:N_CAT]
        idx_v, tmp_v, gsem, wsem = refs[N_CAT:]
        wid = lax.axis_index("s") * NC + lax.axis_index("c")
        pltpu.sync_copy(idx_hbm.at[wid], idx_v)

        pending = [None, None]
        for it in range(N_CAT * 2):
            j, h = it // 2, it % 2
            bslot = it % 2
            if pending[bslot] is not None:
                pending[bslot].wait()
            gcs = [
                pltpu.async_copy(
                    table_hbm.at[idx_v.at[j, 2 * h + s]],
                    tmp_v.at[bslot, pl.ds(s * SUB, SUB)],
                    gsem,
                )
                for s in range(2)
            ]
            for cp in gcs:
                cp.wait()
            pending[bslot] = pltpu.async_copy(
                tmp_v.at[bslot],
                slabs[j].at[pl.ds(wid * WROWS + h * HCH, HCH), :],
                wsem,
            )
        for cp in pending:
            if cp is not None:
                cp.wait()

    return k(table_pad, idx4)


def _assemble_body(xp_ref, s_ref, w2_ref, b2_ref, *refs):
    g_refs = refs[:N_CAT]
    out_ref = refs[N_CAT]
    num = (
        jnp.dot(xp_ref[...], s_ref[...], preferred_element_type=jnp.float32)
        * w2_ref[...]
        + b2_ref[...]
    )
    parts = [num] + [g_refs[j][...][:, :D] for j in range(N_CAT)]
    out_ref[...] = jnp.concatenate(parts, axis=1)


def _assemble(xp, sel, w2, b2, gs):
    bb = 512
    return pl.pallas_call(
        _assemble_body,
        grid=(B // bb,),
        in_specs=[
            pl.BlockSpec((bb, 16), lambda i: (i, 0)),
            pl.BlockSpec((16, CAT_COL0), lambda i: (0, 0)),
            pl.BlockSpec((1, CAT_COL0), lambda i: (0, 0)),
            pl.BlockSpec((1, CAT_COL0), lambda i: (0, 0)),
        ]
        + [pl.BlockSpec((bb, 128), lambda i: (i, 0)) for _ in range(N_CAT)],
        out_specs=pl.BlockSpec((bb, D_OUT), lambda i: (i, 0)),
        out_shape=jax.ShapeDtypeStruct((B, D_OUT), jnp.float32),
        compiler_params=pltpu.CompilerParams(
            dimension_semantics=("arbitrary",),
        ),
    )(xp, sel, w2, b2, *gs)


def kernel(X_num, X_cat, num_weight, num_bias, cat_tables):
    offs = (jnp.arange(N_CAT, dtype=jnp.int32) * VOCAB)[None, :]
    xo = X_cat.astype(jnp.int32) + offs
    idx4 = xo.reshape(NW, NSUB, SUB, N_CAT).transpose(0, 3, 1, 2)
    table_pad = jnp.pad(cat_tables.reshape(T_ROWS, D), ((0, 0), (0, 128 - D)))
    gs = _sc_gather(table_pad, idx4)

    xp = jnp.concatenate(
        [jnp.zeros((B, 1), jnp.float32), X_num, jnp.zeros((B, 2), jnp.float32)],
        axis=1,
    )
    tok_of_col = jnp.arange(CAT_COL0, dtype=jnp.int32) // D
    sel = (tok_of_col[None, :] == jnp.arange(16, dtype=jnp.int32)[:, None]).astype(
        jnp.float32
    )
    w2 = jnp.concatenate([jnp.zeros((D,), jnp.float32), num_weight.reshape(-1)])[
        None, :
    ]
    b2 = jnp.concatenate([jnp.zeros((D,), jnp.float32), num_bias.reshape(-1)])[
        None, :
    ]
    return _assemble(xp, sel, w2, b2, gs)


# speedup vs baseline: 1.0563x; 1.0231x over previous
import functools

import jax
import jax.numpy as jnp
from jax import lax
from jax.experimental import pallas as pl
from jax.experimental.pallas import tpu as pltpu
from jax.experimental.pallas import tpu_sc as plsc

B = 16384
N_NUM = 13
N_CAT = 26
VOCAB = 100000
D = 32
N_TOK = 1 + N_NUM + N_CAT
D_OUT = N_TOK * D
CAT_COL0 = (1 + N_NUM) * D

NG = 7
NC, NS = 2, 16
NW = NC * NS
CHUNK = 1024
N_CHUNKS = B // CHUNK
N_ITEMS = N_CAT * N_CHUNKS
ITEMS_PER_W = N_ITEMS // NW
SUB = 128
NSUB = CHUNK // SUB

VWIN = 2048
VWINS = -(-VOCAB // VWIN)
VPAD = VWINS * VWIN
R_ROWS = N_CAT * VPAD // 4


def _repack_body(t_ref, out_ref):
    x = t_ref[...][0]
    y = x.T.reshape(VWIN // 4, 4, D)
    for a in range(4):
        out_ref[:, pl.ds(a * D, D)] = y[:, a, :]


def _repack(tabT):
    grid = (N_CAT, VWINS)
    return pl.pallas_call(
        _repack_body,
        grid=grid,
        in_specs=[
            pl.BlockSpec((1, D, VWIN), lambda j, v: (j, 0, v)),
        ],
        out_specs=pl.BlockSpec(
            (VWIN // 4, 128), lambda j, v: (j * VWINS + v, 0)
        ),
        out_shape=jax.ShapeDtypeStruct((R_ROWS, 128), jnp.float32),
        compiler_params=pltpu.CompilerParams(
            dimension_semantics=("arbitrary", "arbitrary"),
        ),
    )(tabT)


def _sc_gather(table_flat, idx3):
    mesh = plsc.VectorSubcoreMesh(core_axis_name="c", subcore_axis_name="s")

    @functools.partial(
        pl.kernel,
        out_type=[jax.ShapeDtypeStruct((B, 128), jnp.float32) for _ in range(NG)],
        mesh=mesh,
        scratch_types=[
            pltpu.VMEM((NSUB, SUB), jnp.int32),
            pltpu.VMEM((CHUNK, D), jnp.float32),
            pltpu.SemaphoreType.DMA,
        ],
        compiler_params=pltpu.CompilerParams(use_tc_tiling_on_sc=False),
    )
    def k(table_hbm, idx_hbm, *refs):
        g_refs = refs[:NG]
        idx_v, tmp_v, gsem = refs[NG:]
        wid = lax.axis_index("s") * NC + lax.axis_index("c")

        @pl.loop(0, ITEMS_PER_W)
        def _(it):
            item = wid * ITEMS_PER_W + it
            j = item // N_CHUNKS
            c = item - j * N_CHUNKS
            pltpu.sync_copy(idx_hbm.at[item], idx_v)
            cps = [
                pltpu.async_copy(
                    table_hbm.at[idx_v.at[s]],
                    tmp_v.at[pl.ds(s * SUB, SUB)],
                    gsem,
                )
                for s in range(NSUB)
            ]
            for cp in cps:
                cp.wait()
            gi = j // 4
            f = j - gi * 4
            dst = [
                g.at[pl.ds(c * CHUNK, CHUNK), pl.ds(f * D, D)] for g in g_refs
            ]
            for gg in range(NG):
                @pl.when(gi == gg)
                def _():
                    pltpu.sync_copy(tmp_v, dst[gg])

    return k(table_flat, idx3)


def _assemble_body(xp_ref, s_ref, w2_ref, b2_ref, *refs):
    g_refs = refs[:NG]
    out_ref = refs[NG]
    num = (
        jnp.dot(xp_ref[...], s_ref[...], preferred_element_type=jnp.float32)
        * w2_ref[...]
        + b2_ref[...]
    )
    parts = [num] + [g_refs[gi][...] for gi in range(NG - 1)]
    parts.append(g_refs[NG - 1][...][:, : 2 * D])
    out_ref[...] = jnp.concatenate(parts, axis=1)


def _assemble(xp, sel, w2, b2, gs):
    bb = 512
    return pl.pallas_call(
        _assemble_body,
        grid=(B // bb,),
        in_specs=[
            pl.BlockSpec((bb, 16), lambda i: (i, 0)),
            pl.BlockSpec((16, CAT_COL0), lambda i: (0, 0)),
            pl.BlockSpec((1, CAT_COL0), lambda i: (0, 0)),
            pl.BlockSpec((1, CAT_COL0), lambda i: (0, 0)),
        ]
        + [pl.BlockSpec((bb, 128), lambda i: (i, 0)) for _ in range(NG)],
        out_specs=pl.BlockSpec((bb, D_OUT), lambda i: (i, 0)),
        out_shape=jax.ShapeDtypeStruct((B, D_OUT), jnp.float32),
        compiler_params=pltpu.CompilerParams(
            dimension_semantics=("arbitrary",),
        ),
    )(xp, sel, w2, b2, *gs)


def kernel(X_num, X_cat, num_weight, num_bias, cat_tables):
    offs = (jnp.arange(N_CAT, dtype=jnp.int32) * VPAD)[None, :]
    xo = X_cat.astype(jnp.int32) + offs
    idx3 = (
        xo.reshape(N_CHUNKS, NSUB, SUB, N_CAT)
        .transpose(3, 0, 1, 2)
        .reshape(N_ITEMS, NSUB, SUB)
    )
    tabT = cat_tables.transpose(0, 2, 1)
    table_flat = _repack(tabT).reshape(N_CAT * VPAD, D)
    gs = _sc_gather(table_flat, idx3)

    xp = jnp.concatenate(
        [jnp.zeros((B, 1), jnp.float32), X_num, jnp.zeros((B, 2), jnp.float32)],
        axis=1,
    )
    tok_of_col = jnp.arange(CAT_COL0, dtype=jnp.int32) // D
    sel = (tok_of_col[None, :] == jnp.arange(16, dtype=jnp.int32)[:, None]).astype(
        jnp.float32
    )
    w2 = jnp.concatenate([jnp.zeros((D,), jnp.float32), num_weight.reshape(-1)])[
        None, :
    ]
    b2 = jnp.concatenate([jnp.zeros((D,), jnp.float32), num_bias.reshape(-1)])[
        None, :
    ]
    return _assemble(xp, sel, w2, b2, gs)


# speedup vs baseline: 1.1892x; 1.1258x over previous
import functools

import jax
import jax.numpy as jnp
from jax import lax
from jax.experimental import pallas as pl
from jax.experimental.pallas import tpu as pltpu
from jax.experimental.pallas import tpu_sc as plsc

B = 16384
N_NUM = 13
N_CAT = 26
VOCAB = 100000
D = 32
N_TOK = 1 + N_NUM + N_CAT
D_OUT = N_TOK * D
CAT_COL0 = (1 + N_NUM) * D

NG = 7
NC, NS = 2, 16
NW = NC * NS
CHUNK = 1024
N_CHUNKS = B // CHUNK
N_ITEMS = N_CAT * N_CHUNKS
ITEMS_PER_W = N_ITEMS // NW
SUB = 128
NSUB = CHUNK // SUB

T_ROWS = N_CAT * VOCAB
BBR = 8000
R_ROWS = T_ROWS // 4


def _detile_body(t_ref, out_ref):
    y = t_ref[...].reshape(BBR // 4, 4, D)
    for a in range(4):
        out_ref[:, pl.ds(a * D, D)] = y[:, a, :]


def _detile(table_rm):
    return pl.pallas_call(
        _detile_body,
        grid=(T_ROWS // BBR,),
        in_specs=[pl.BlockSpec((BBR, D), lambda i: (i, 0))],
        out_specs=pl.BlockSpec((BBR // 4, 128), lambda i: (i, 0)),
        out_shape=jax.ShapeDtypeStruct((R_ROWS, 128), jnp.float32),
        compiler_params=pltpu.CompilerParams(
            dimension_semantics=("arbitrary",),
        ),
    )(table_rm)


def _sc_gather(table_flat, idx3):
    mesh = plsc.VectorSubcoreMesh(core_axis_name="c", subcore_axis_name="s")

    @functools.partial(
        pl.kernel,
        out_type=[jax.ShapeDtypeStruct((B, 128), jnp.float32) for _ in range(NG)],
        mesh=mesh,
        scratch_types=[
            pltpu.VMEM((NSUB, SUB), jnp.int32),
            pltpu.VMEM((CHUNK, D), jnp.float32),
            pltpu.SemaphoreType.DMA,
        ],
        compiler_params=pltpu.CompilerParams(use_tc_tiling_on_sc=False),
    )
    def k(table_hbm, idx_hbm, *refs):
        g_refs = refs[:NG]
        idx_v, tmp_v, gsem = refs[NG:]
        wid = lax.axis_index("s") * NC + lax.axis_index("c")

        @pl.loop(0, ITEMS_PER_W)
        def _(it):
            item = wid * ITEMS_PER_W + it
            j = item // N_CHUNKS
            c = item - j * N_CHUNKS
            pltpu.sync_copy(idx_hbm.at[item], idx_v)
            cps = [
                pltpu.async_copy(
                    table_hbm.at[idx_v.at[s]],
                    tmp_v.at[pl.ds(s * SUB, SUB)],
                    gsem,
                )
                for s in range(NSUB)
            ]
            for cp in cps:
                cp.wait()
            gi = j // 4
            f = j - gi * 4
            dst = [
                g.at[pl.ds(c * CHUNK, CHUNK), pl.ds(f * D, D)] for g in g_refs
            ]
            for gg in range(NG):
                @pl.when(gi == gg)
                def _():
                    pltpu.sync_copy(tmp_v, dst[gg])

    return k(table_flat, idx3)


def _assemble_body(xp_ref, s_ref, w2_ref, b2_ref, *refs):
    g_refs = refs[:NG]
    out_ref = refs[NG]
    num = (
        jnp.dot(xp_ref[...], s_ref[...], preferred_element_type=jnp.float32)
        * w2_ref[...]
        + b2_ref[...]
    )
    parts = [num] + [g_refs[gi][...] for gi in range(NG - 1)]
    parts.append(g_refs[NG - 1][...][:, : 2 * D])
    out_ref[...] = jnp.concatenate(parts, axis=1)


def _assemble(xp, sel, w2, b2, gs):
    bb = 512
    return pl.pallas_call(
        _assemble_body,
        grid=(B // bb,),
        in_specs=[
            pl.BlockSpec((bb, 16), lambda i: (i, 0)),
            pl.BlockSpec((16, CAT_COL0), lambda i: (0, 0)),
            pl.BlockSpec((1, CAT_COL0), lambda i: (0, 0)),
            pl.BlockSpec((1, CAT_COL0), lambda i: (0, 0)),
        ]
        + [pl.BlockSpec((bb, 128), lambda i: (i, 0)) for _ in range(NG)],
        out_specs=pl.BlockSpec((bb, D_OUT), lambda i: (i, 0)),
        out_shape=jax.ShapeDtypeStruct((B, D_OUT), jnp.float32),
        compiler_params=pltpu.CompilerParams(
            dimension_semantics=("arbitrary",),
        ),
    )(xp, sel, w2, b2, *gs)


def kernel(X_num, X_cat, num_weight, num_bias, cat_tables):
    offs = (jnp.arange(N_CAT, dtype=jnp.int32) * VOCAB)[None, :]
    xo = X_cat.astype(jnp.int32) + offs
    idx3 = (
        xo.reshape(N_CHUNKS, NSUB, SUB, N_CAT)
        .transpose(3, 0, 1, 2)
        .reshape(N_ITEMS, NSUB, SUB)
    )
    table_rm = cat_tables.reshape(T_ROWS, D)
    table_flat = _detile(table_rm).reshape(T_ROWS, D)
    gs = _sc_gather(table_flat, idx3)

    xp = jnp.concatenate(
        [jnp.zeros((B, 1), jnp.float32), X_num, jnp.zeros((B, 2), jnp.float32)],
        axis=1,
    )
    tok_of_col = jnp.arange(CAT_COL0, dtype=jnp.int32) // D
    sel = (tok_of_col[None, :] == jnp.arange(16, dtype=jnp.int32)[:, None]).astype(
        jnp.float32
    )
    w2 = jnp.concatenate([jnp.zeros((D,), jnp.float32), num_weight.reshape(-1)])[
        None, :
    ]
    b2 = jnp.concatenate([jnp.zeros((D,), jnp.float32), num_bias.reshape(-1)])[
        None, :
    ]
    return _assemble(xp, sel, w2, b2, gs)


# speedup vs baseline: 1.3086x; 1.1005x over previous
import functools

import jax
import jax.numpy as jnp
from jax import lax
from jax.experimental import pallas as pl
from jax.experimental.pallas import tpu as pltpu
from jax.experimental.pallas import tpu_sc as plsc

B = 16384
N_NUM = 13
N_CAT = 26
VOCAB = 100000
D = 32
N_TOK = 1 + N_NUM + N_CAT
D_OUT = N_TOK * D
CAT_COL0 = (1 + N_NUM) * D

NG = 7
NC, NS = 2, 16
NW = NC * NS
CHUNK = 1024
N_CHUNKS = B // CHUNK
N_ITEMS = N_CAT * N_CHUNKS
ITEMS_PER_W = N_ITEMS // NW
SUB = 128
NSUB = CHUNK // SUB

T_ROWS = N_CAT * VOCAB
BBR = 20000
R_ROWS = T_ROWS // 4


def _detile_body(t_ref, out_ref):
    y = t_ref[...].reshape(BBR // 4, 4, D)
    for a in range(4):
        out_ref[:, pl.ds(a * D, D)] = y[:, a, :]


def _detile(table_rm):
    return pl.pallas_call(
        _detile_body,
        grid=(T_ROWS // BBR,),
        in_specs=[pl.BlockSpec((BBR, D), lambda i: (i, 0))],
        out_specs=pl.BlockSpec((BBR // 4, 128), lambda i: (i, 0)),
        out_shape=jax.ShapeDtypeStruct((R_ROWS, 128), jnp.float32),
        compiler_params=pltpu.CompilerParams(
            dimension_semantics=("arbitrary",),
        ),
    )(table_rm)


def _sc_gather(table_flat, idx3):
    mesh = plsc.VectorSubcoreMesh(core_axis_name="c", subcore_axis_name="s")

    @functools.partial(
        pl.kernel,
        out_type=[jax.ShapeDtypeStruct((B, 128), jnp.float32) for _ in range(NG)],
        mesh=mesh,
        scratch_types=[
            pltpu.VMEM((NSUB, SUB), jnp.int32),
            pltpu.VMEM((CHUNK, D), jnp.float32),
            pltpu.SemaphoreType.DMA,
        ],
        compiler_params=pltpu.CompilerParams(use_tc_tiling_on_sc=False),
    )
    def k(table_hbm, idx_hbm, *refs):
        g_refs = refs[:NG]
        idx_v, tmp_v, gsem = refs[NG:]
        wid = lax.axis_index("s") * NC + lax.axis_index("c")

        @pl.loop(0, ITEMS_PER_W)
        def _(it):
            item = wid * ITEMS_PER_W + it
            j = item // N_CHUNKS
            c = item - j * N_CHUNKS
            pltpu.sync_copy(idx_hbm.at[item], idx_v)
            cps = [
                pltpu.async_copy(
                    table_hbm.at[idx_v.at[s]],
                    tmp_v.at[pl.ds(s * SUB, SUB)],
                    gsem,
                )
                for s in range(NSUB)
            ]
            for cp in cps:
                cp.wait()
            gi = j // 4
            f = j - gi * 4
            dst = [
                g.at[pl.ds(c * CHUNK, CHUNK), pl.ds(f * D, D)] for g in g_refs
            ]
            for gg in range(NG):
                @pl.when(gi == gg)
                def _():
                    pltpu.sync_copy(tmp_v, dst[gg])

    return k(table_flat, idx3)


def _assemble_body(xp_ref, s_ref, w2_ref, b2_ref, *refs):
    g_refs = refs[:NG]
    out_ref = refs[NG]
    num = (
        jnp.dot(xp_ref[...], s_ref[...], preferred_element_type=jnp.float32)
        * w2_ref[...]
        + b2_ref[...]
    )
    parts = [num] + [g_refs[gi][...] for gi in range(NG - 1)]
    parts.append(g_refs[NG - 1][...][:, : 2 * D])
    out_ref[...] = jnp.concatenate(parts, axis=1)


def _assemble(xp, sel, w2, b2, gs):
    bb = 512
    return pl.pallas_call(
        _assemble_body,
        grid=(B // bb,),
        in_specs=[
            pl.BlockSpec((bb, 16), lambda i: (i, 0)),
            pl.BlockSpec((16, CAT_COL0), lambda i: (0, 0)),
            pl.BlockSpec((1, CAT_COL0), lambda i: (0, 0)),
            pl.BlockSpec((1, CAT_COL0), lambda i: (0, 0)),
        ]
        + [pl.BlockSpec((bb, 128), lambda i: (i, 0)) for _ in range(NG)],
        out_specs=pl.BlockSpec((bb, D_OUT), lambda i: (i, 0)),
        out_shape=jax.ShapeDtypeStruct((B, D_OUT), jnp.float32),
        compiler_params=pltpu.CompilerParams(
            dimension_semantics=("arbitrary",),
        ),
    )(xp, sel, w2, b2, *gs)


def kernel(X_num, X_cat, num_weight, num_bias, cat_tables):
    offs = (jnp.arange(N_CAT, dtype=jnp.int32) * VOCAB)[None, :]
    xo = X_cat.astype(jnp.int32) + offs
    idx3 = (
        xo.reshape(N_CHUNKS, NSUB, SUB, N_CAT)
        .transpose(3, 0, 1, 2)
        .reshape(N_ITEMS, NSUB, SUB)
    )
    table_rm = cat_tables.reshape(T_ROWS, D)
    table_flat = _detile(table_rm).reshape(T_ROWS, D)
    gs = _sc_gather(table_flat, idx3)

    xp = jnp.concatenate(
        [jnp.zeros((B, 1), jnp.float32), X_num, jnp.zeros((B, 2), jnp.float32)],
        axis=1,
    )
    tok_of_col = jnp.arange(CAT_COL0, dtype=jnp.int32) // D
    sel = (tok_of_col[None, :] == jnp.arange(16, dtype=jnp.int32)[:, None]).astype(
        jnp.float32
    )
    w2 = jnp.concatenate([jnp.zeros((D,), jnp.float32), num_weight.reshape(-1)])[
        None, :
    ]
    b2 = jnp.concatenate([jnp.zeros((D,), jnp.float32), num_bias.reshape(-1)])[
        None, :
    ]
    return _assemble(xp, sel, w2, b2, gs)
